# baseline (speedup 1.0000x reference)
import jax
import jax.numpy as jnp
from jax.experimental import pallas as pl
from jax.experimental.pallas import tpu as pltpu

EPS = 1e-3
FORGET_BIAS = 1.0


def _ln(v, gamma, beta):
    mean = jnp.mean(v, axis=1, keepdims=True)
    vc = v - mean
    var = jnp.mean(vc * vc, axis=1, keepdims=True)
    return gamma * (vc * jax.lax.rsqrt(var + EPS)) + beta


def _lstm_kernel(x_ref, c_ref, h_ref, wx_ref, wh_ref, b_ref, g_ref, be_ref,
                 gc_ref, bc_ref, h_out_ref, c_out_ref):
    H = c_ref.shape[1]
    x = x_ref[...]
    h = h_ref[...]
    def gate(g):
        sl = slice(g * H, (g + 1) * H)
        acc = jnp.dot(x, wx_ref[:, sl], preferred_element_type=jnp.float32)
        acc = acc + jnp.dot(h, wh_ref[:, sl], preferred_element_type=jnp.float32)
        acc = acc + b_ref[:, sl]
        return _ln(acc, g_ref[:, sl], be_ref[:, sl])

    si = jax.nn.sigmoid(gate(0))
    tj = jnp.tanh(gate(1))
    sf = jax.nn.sigmoid(gate(2) + FORGET_BIAS)
    c = c_ref[...]
    new_c = c * sf + si * tj
    c_out_ref[...] = new_c
    c_ln = _ln(new_c, gc_ref[...], bc_ref[...])
    th = jnp.tanh(c_ln)
    h_out_ref[...] = th * jax.nn.sigmoid(gate(3))


def kernel(x, c, h, W_xh, W_hh, bias, ln_gamma, ln_beta, ln_c_gamma, ln_c_beta):
    B, I = x.shape
    H = c.shape[1]
    BB = min(512, B)
    nb = B // BB

    b2 = bias.reshape(1, 4 * H)
    g2 = ln_gamma.reshape(1, 4 * H)
    be2 = ln_beta.reshape(1, 4 * H)
    gc2 = ln_c_gamma.reshape(1, H)
    bc2 = ln_c_beta.reshape(1, H)

    row = lambda i: (i, 0)
    fixed = lambda i: (0, 0)
    new_h, new_c = pl.pallas_call(
        _lstm_kernel,
        grid=(nb,),
        in_specs=[
            pl.BlockSpec((BB, I), row),
            pl.BlockSpec((BB, H), row),
            pl.BlockSpec((BB, H), row),
            pl.BlockSpec((I, 4 * H), fixed),
            pl.BlockSpec((H, 4 * H), fixed),
            pl.BlockSpec((1, 4 * H), fixed),
            pl.BlockSpec((1, 4 * H), fixed),
            pl.BlockSpec((1, 4 * H), fixed),
            pl.BlockSpec((1, H), fixed),
            pl.BlockSpec((1, H), fixed),
        ],
        out_specs=[
            pl.BlockSpec((BB, H), row),
            pl.BlockSpec((BB, H), row),
        ],
        out_shape=[
            jax.ShapeDtypeStruct((B, H), jnp.float32),
            jax.ShapeDtypeStruct((B, H), jnp.float32),
        ],
        compiler_params=pltpu.CompilerParams(
            dimension_semantics=("parallel",),
            vmem_limit_bytes=100 * 1024 * 1024,
        ),
        name="lstm_cell_fused",
    )(x, c, h, W_xh, W_hh, b2, g2, be2, gc2, bc2)
    return new_h, new_c
